# baseline (speedup 1.0000x reference)
import functools

import jax
import jax.numpy as jnp
from jax import lax
from jax.experimental import pallas as pl
from jax.experimental.pallas import tpu as pltpu
from jax.experimental.pallas import tpu_sc as plsc

_D = 64
_B = 4096 * 200
_NC, _NS = 2, 16
_NW = _NC * _NS
_BW = _B // _NW
_ROWS = 4096
_SEQ = 200
_CHUNK = 512
_BBLK = _ROWS // _CHUNK
_CHUNKS = _BW // _CHUNK
_NSLOT = 3
_LAG = 2


def _body(idx_hbm, table_hbm, out_hbm, idx_v, *bufs_sems):
    bufs = bufs_sems[:_NSLOT]
    gsems = bufs_sems[_NSLOT : 2 * _NSLOT]
    wsems = bufs_sems[2 * _NSLOT :]
    wid = lax.axis_index("s") * _NC + lax.axis_index("c")
    base = wid * _CHUNKS

    pltpu.sync_copy(idx_hbm.at[wid], idx_v)

    def gather(c, b):
        return pltpu.make_async_copy(table_hbm.at[idx_v.at[c]], bufs[b], gsems[b])

    def write(c, b):
        g = base + c
        s = g // _BBLK
        bb = g % _BBLK
        return pltpu.make_async_copy(
            bufs[b], out_hbm.at[s, pl.ds(bb * _CHUNK, _CHUNK)], wsems[b]
        )

    def visit(v, b):
        @pl.when(jnp.logical_and(v >= _NSLOT, v - _NSLOT < _CHUNKS))
        def _():
            write(v - _NSLOT, b).wait()

        @pl.when(v < _CHUNKS)
        def _():
            gather(v, b).start()

        b2 = (b - _LAG) % _NSLOT

        @pl.when(jnp.logical_and(v >= _LAG, v - _LAG < _CHUNKS))
        def _():
            gather(v - _LAG, b2).wait()
            write(v - _LAG, b2).start()

    def outer(g, carry):
        v0 = g * _NSLOT
        for j in range(_NSLOT):
            visit(v0 + j, j)
        return carry

    lax.fori_loop(0, (_CHUNKS + _NSLOT) // _NSLOT + 1, outer, 0)


@functools.partial(
    pl.kernel,
    out_type=jax.ShapeDtypeStruct((_SEQ, _ROWS, _D), jnp.float32),
    mesh=plsc.VectorSubcoreMesh(
        core_axis_name="c", subcore_axis_name="s", num_cores=_NC, num_subcores=_NS
    ),
    scratch_types=(
        [pltpu.VMEM((_CHUNKS, _CHUNK), jnp.int32)]
        + [pltpu.VMEM((_CHUNK, _D), jnp.float32) for _ in range(_NSLOT)]
        + [pltpu.SemaphoreType.DMA for _ in range(2 * _NSLOT)]
    ),
    compiler_params=pltpu.CompilerParams(use_tc_tiling_on_sc=False),
)
def _gather_rows(idx_hbm, table_hbm, out_hbm, idx_v, *bufs_sems):
    _body(idx_hbm, table_hbm, out_hbm, idx_v, *bufs_sems)


def kernel(input, table):
    idx = input.T.reshape(_NW, _CHUNKS, _CHUNK).astype(jnp.int32)
    out = _gather_rows(idx, table)
    return out.transpose(1, 0, 2)
